# Initial kernel scaffold; baseline (speedup 1.0000x reference)
import jax
import jax.numpy as jnp
from jax.experimental import pallas as pl


def kernel(x, attn, W1, b1, W2, b2, Wc, bc):
    raise NotImplementedError("write your pallas kernel here")


# speedup vs baseline: 71.9849x; 71.9849x over previous
import functools

import jax
import jax.numpy as jnp
from jax.experimental import pallas as pl

_B = 4
_TS = 288
_DIM = 768
_K = int(_TS * _TS * 0.25)


def _gnn_body(a_ref, nodes_ref, w1_ref, b1_ref, w2_ref, b2_ref, wd_ref,
              bd_ref, out_ref):
    A = a_ref[0]
    nodes = nodes_ref[0]

    bits = jax.lax.bitcast_convert_type(A, jnp.int32)
    res = jnp.int32(0)
    for bit in range(30, -1, -1):
        cand = res | jnp.int32(1 << bit)
        cnt = jnp.sum(jnp.where(bits >= cand, jnp.int32(1), jnp.int32(0)))
        res = jnp.where(cnt >= _K, cand, res)

    S = jnp.where(bits >= res, A, 0.0)
    S_T = S.T
    deg = 1.0 + jnp.sum(S_T, axis=1, keepdims=True)
    dinv = 1.0 / jnp.sqrt(deg)
    dinv2 = dinv * dinv

    def gcn(h, w_ref, b_ref):
        xw = jnp.dot(h, w_ref[...], preferred_element_type=jnp.float32)
        agg = jnp.dot(S_T, dinv * xw, preferred_element_type=jnp.float32)
        return dinv * agg + dinv2 * xw + b_ref[...]

    h1 = jnp.maximum(gcn(nodes, w1_ref, b1_ref), 0.0)
    h2 = jnp.maximum(gcn(h1, w2_ref, b2_ref), 0.0)

    d = jnp.sum(h2 * wd_ref[...], axis=1, keepdims=True) + bd_ref[0, 0]
    p1 = 1.0 / (1.0 + jnp.exp(-d))
    p0 = 1.0 - p1
    out_ref[0, 0:1, :] = jnp.sum(p0 * nodes, axis=0, keepdims=True)
    out_ref[0, 1:2, :] = jnp.sum(p1 * nodes, axis=0, keepdims=True)


@jax.jit
def kernel(x, attn, W1, b1, W2, b2, Wc, bc):
    n = _TS
    non_skip_tk = x[:, 1:1 + n]
    skip_tk = x[:, 1 + n:]
    A = attn[:, 1 + n:, 1 + n:]

    wd = (Wc[:, 1] - Wc[:, 0]).reshape(1, _DIM)
    bd = (bc[1] - bc[0]).reshape(1, 1)

    summaries = pl.pallas_call(
        _gnn_body,
        grid=(_B,),
        in_specs=[
            pl.BlockSpec((1, _TS, _TS), lambda b: (b, 0, 0)),
            pl.BlockSpec((1, _TS, _DIM), lambda b: (b, 0, 0)),
            pl.BlockSpec((_DIM, _DIM), lambda b: (0, 0)),
            pl.BlockSpec((1, _DIM), lambda b: (0, 0)),
            pl.BlockSpec((_DIM, _DIM), lambda b: (0, 0)),
            pl.BlockSpec((1, _DIM), lambda b: (0, 0)),
            pl.BlockSpec((1, _DIM), lambda b: (0, 0)),
            pl.BlockSpec((1, 1), lambda b: (0, 0)),
        ],
        out_specs=pl.BlockSpec((1, 2, _DIM), lambda b: (b, 0, 0)),
        out_shape=jax.ShapeDtypeStruct((_B, 2, _DIM), jnp.float32),
    )(A, skip_tk, W1, b1.reshape(1, _DIM), W2, b2.reshape(1, _DIM), wd, bd)

    return jnp.concatenate([non_skip_tk, summaries], axis=1)


# speedup vs baseline: 109.2251x; 1.5173x over previous
import jax
import jax.numpy as jnp
from jax.experimental import pallas as pl

_B = 4
_TS = 288
_DIM = 768
_K = int(_TS * _TS * 0.25)


def _gnn_body(at_ref, nodes_ref, w1_ref, b1_ref, w2_ref, b2_ref, wd_ref,
              bd_ref, out_ref):
    AT = at_ref[...]
    nodes = nodes_ref[...].reshape(_B * _TS, _DIM)

    bits = jax.lax.bitcast_convert_type(AT, jnp.int32)
    res = jnp.zeros((_B, 1, 1), jnp.int32)
    for bit in range(30, -1, -1):
        cand = res | jnp.int32(1 << bit)
        m = jnp.where(bits >= cand, jnp.int32(1), jnp.int32(0))
        cnt = jnp.sum(m, axis=(1, 2), keepdims=True)
        res = jnp.where(cnt >= _K, cand, res)

    S_T = jnp.where(bits >= res, AT, 0.0)
    deg = 1.0 + jnp.sum(S_T, axis=2, keepdims=True)
    dinv3 = 1.0 / jnp.sqrt(deg)
    dinv = dinv3.reshape(_B * _TS, 1)
    dinv2 = dinv * dinv

    def gcn(h, w_ref, b_ref):
        xw = jnp.dot(h, w_ref[...], preferred_element_type=jnp.float32)
        y = (dinv * xw).reshape(_B, _TS, _DIM)
        agg = jax.lax.dot_general(
            S_T, y, (((2,), (1,)), ((0,), (0,))),
            preferred_element_type=jnp.float32).reshape(_B * _TS, _DIM)
        return dinv * agg + dinv2 * xw + b_ref[...]

    h1 = jnp.maximum(gcn(nodes, w1_ref, b1_ref), 0.0)
    h2 = jnp.maximum(gcn(h1, w2_ref, b2_ref), 0.0)

    d = jnp.sum(h2 * wd_ref[...], axis=1, keepdims=True) + bd_ref[0, 0]
    p1 = 1.0 / (1.0 + jnp.exp(-d))
    p0 = 1.0 - p1
    nodes3 = nodes.reshape(_B, _TS, _DIM)
    r0 = jnp.sum(p0.reshape(_B, _TS, 1) * nodes3, axis=1, keepdims=True)
    r1 = jnp.sum(p1.reshape(_B, _TS, 1) * nodes3, axis=1, keepdims=True)
    out_ref[...] = jnp.concatenate([r0, r1], axis=1)


@jax.jit
def kernel(x, attn, W1, b1, W2, b2, Wc, bc):
    n = _TS
    non_skip_tk = x[:, 1:1 + n]
    skip_tk = x[:, 1 + n:]
    A_T = jnp.swapaxes(attn[:, 1 + n:, 1 + n:], 1, 2)

    wd = (Wc[:, 1] - Wc[:, 0]).reshape(1, _DIM)
    bd = (bc[1] - bc[0]).reshape(1, 1)

    summaries = pl.pallas_call(
        _gnn_body,
        out_shape=jax.ShapeDtypeStruct((_B, 2, _DIM), jnp.float32),
    )(A_T, skip_tk, W1, b1.reshape(1, _DIM), W2, b2.reshape(1, _DIM), wd, bd)

    return jnp.concatenate([non_skip_tk, summaries], axis=1)
